# Initial kernel scaffold; baseline (speedup 1.0000x reference)
import jax
import jax.numpy as jnp
from jax.experimental import pallas as pl


def kernel(g, node_feat, edge_feat, embed_weight, W_self, W_neigh, w_gate, b):
    raise NotImplementedError("write your pallas kernel here")


# speedup vs baseline: 5.1068x; 5.1068x over previous
import functools

import jax
import jax.numpy as jnp
from jax import lax
from jax.experimental import pallas as pl
from jax.experimental.pallas import tpu as pltpu
from jax.experimental.pallas import tpu_sc as plsc

_NC = 2
_NS = 16
_LANES = 16


def _lane_bcast(v, l):
    idx = jnp.full((_LANES, 1), l, jnp.int32)
    dn = lax.GatherDimensionNumbers(
        offset_dims=(), collapsed_slice_dims=(0,), start_index_map=(0,))
    return lax.gather(v, idx, dn, slice_sizes=(1,),
                      mode=lax.GatherScatterMode.PROMISE_IN_BOUNDS)


def _gate_body(ef_ref, w_ref, out_ref):
    z = jnp.dot(ef_ref[...], w_ref[...], preferred_element_type=jnp.float32)
    out_ref[...] = jax.nn.sigmoid(z)


def _compute_gate(edge_feat, w_gate):
    E, DE = edge_feat.shape
    P = 128
    R = E // P
    ef2 = edge_feat.reshape(R, P * DE)
    wbig = jnp.kron(jnp.eye(P, dtype=jnp.float32), w_gate.reshape(DE, 1))
    out = pl.pallas_call(
        _gate_body,
        out_shape=jax.ShapeDtypeStruct((R, P), jnp.float32),
    )(ef2, wbig)
    return out.reshape(E)


def _make_sc_agg(N, D, E):
    NW = _NC * _NS
    EW = E // NW
    C = 80
    CH = EW // C
    RT = 624
    ZR = 48
    TAIL = N - _NS * RT
    GRP = C // _LANES
    JD = D // _LANES
    assert EW * NW == E and CH * C == EW and CH % 2 == 1
    assert RT == 13 * ZR and TAIL == 16

    mesh = plsc.VectorSubcoreMesh(core_axis_name="c", subcore_axis_name="s")

    @functools.partial(
        pl.kernel,
        out_type=jax.ShapeDtypeStruct((_NC, N, D), jnp.float32),
        mesh=mesh,
        scratch_types=[
            pltpu.VMEM((EW,), jnp.int32),
            pltpu.VMEM((EW,), jnp.float32),
            pltpu.VMEM((C,), jnp.int32),
            pltpu.VMEM((C,), jnp.int32),
            pltpu.VMEM((C, D), jnp.float32),
            pltpu.VMEM((C, D), jnp.float32),
            pltpu.VMEM((ZR, D), jnp.float32),
            pltpu.VMEM_SHARED((N, D), jnp.float32),
            pltpu.SemaphoreType.DMA,
            pltpu.SemaphoreType.DMA,
            pltpu.SemaphoreType.DMA,
            pltpu.SemaphoreType.DMA,
            pltpu.SemaphoreType.DMA,
            pltpu.SemaphoreType.DMA,
            pltpu.SemaphoreType.DMA,
        ],
    )
    def sc_agg(src_hbm, dst_hbm, gate_hbm, x_hbm, out_hbm,
               src_all, gate_all, dstv0, dstv1, rows0, rows1,
               zbuf, agg_sh, sem_i, sem_g0, sem_g1, sem_s0, sem_s1,
               sem_d0, sem_d1):
        c = lax.axis_index("c")
        s = lax.axis_index("s")
        wid = s * _NC + c
        base_w = wid * EW

        cp_src = pltpu.async_copy(src_hbm.at[pl.ds(base_w, EW)], src_all, sem_i)
        cp_gate = pltpu.async_copy(gate_hbm.at[pl.ds(base_w, EW)], gate_all, sem_i)

        zeros = jnp.zeros((_LANES,), jnp.float32)

        def zrow(r, carry):
            for j in range(JD):
                zbuf[r, pl.ds(j * _LANES, _LANES)] = zeros
            return carry

        lax.fori_loop(0, ZR, zrow, 0)
        for i in range(RT // ZR):
            pltpu.sync_copy(zbuf, agg_sh.at[pl.ds(s * RT + i * ZR, ZR)])

        @pl.when(s == 0)
        def _zero_tail():
            pltpu.sync_copy(zbuf.at[pl.ds(0, TAIL)], agg_sh.at[pl.ds(_NS * RT, TAIL)])

        cp_src.wait()
        cp_gate.wait()
        plsc.subcore_barrier()

        def start_gather(k, rows, sem):
            pltpu.async_copy(x_hbm.at[src_all.at[pl.ds(k * C, C)]], rows, sem)

        def wait_gather(k, rows, sem):
            pltpu.make_async_copy(x_hbm.at[src_all.at[pl.ds(k * C, C)]], rows, sem).wait()

        def start_dst(k, dstv, sem):
            pltpu.async_copy(dst_hbm.at[pl.ds(base_w + k * C, C)], dstv, sem)

        def wait_dst(k, dstv, sem):
            pltpu.make_async_copy(dst_hbm.at[pl.ds(base_w + k * C, C)], dstv, sem).wait()

        def start_scatter(rows, dstv, sem):
            pltpu.async_copy(rows, agg_sh.at[dstv], sem, add=True)

        def wait_scatter(rows, dstv, sem):
            pltpu.make_async_copy(rows, agg_sh.at[dstv], sem).wait()

        def scale(k, rows):
            def grp(q, gcarry):
                gv = gate_all[pl.ds(k * C + q * _LANES, _LANES)]
                for l in range(_LANES):
                    g16 = _lane_bcast(gv, l)
                    e = q * _LANES + l
                    for j in range(JD):
                        sl = pl.ds(j * _LANES, _LANES)
                        rows[e, sl] = rows[e, sl] * g16
                return gcarry

            lax.fori_loop(0, GRP, grp, 0)

        start_gather(0, rows0, sem_g0)
        start_dst(0, dstv0, sem_d0)
        start_gather(1, rows1, sem_g1)
        start_dst(1, dstv1, sem_d1)
        wait_gather(0, rows0, sem_g0)
        scale(0, rows0)
        wait_dst(0, dstv0, sem_d0)
        start_scatter(rows0, dstv0, sem_s0)

        def pair(t, carry):
            a = 2 * t + 1
            wait_gather(a, rows1, sem_g1)
            scale(a, rows1)
            wait_dst(a, dstv1, sem_d1)
            start_scatter(rows1, dstv1, sem_s1)
            wait_scatter(rows0, dstv0, sem_s0)
            start_gather(a + 1, rows0, sem_g0)
            start_dst(a + 1, dstv0, sem_d0)
            wait_gather(a + 1, rows0, sem_g0)
            scale(a + 1, rows0)
            wait_dst(a + 1, dstv0, sem_d0)
            start_scatter(rows0, dstv0, sem_s0)
            wait_scatter(rows1, dstv1, sem_s1)

            @pl.when(a + 2 < CH)
            def _next():
                start_gather(a + 2, rows1, sem_g1)
                start_dst(a + 2, dstv1, sem_d1)

            return carry

        lax.fori_loop(0, (CH - 1) // 2, pair, 0)
        wait_scatter(rows0, dstv0, sem_s0)
        plsc.subcore_barrier()

        pltpu.sync_copy(agg_sh.at[pl.ds(s * RT, RT)], out_hbm.at[c, pl.ds(s * RT, RT)])

        @pl.when(s == 0)
        def _write_tail():
            pltpu.sync_copy(agg_sh.at[pl.ds(_NS * RT, TAIL)],
                            out_hbm.at[c, pl.ds(_NS * RT, TAIL)])

    return sc_agg


def _out_body(a0_ref, a1_ref, x_ref, wn_ref, ws_ref, b_ref, o_ref):
    agg = a0_ref[...] + a1_ref[...]
    acc = jnp.dot(agg, wn_ref[...], preferred_element_type=jnp.float32)
    acc = acc + jnp.dot(x_ref[...], ws_ref[...], preferred_element_type=jnp.float32)
    acc = acc + b_ref[...]
    o_ref[...] = jnp.maximum(acc, 0.0)


def _compute_out(a0, a1, x, W_neigh, W_self, b):
    N, D = x.shape
    R = 1000
    G = N // R
    b_row = b.reshape(1, D)
    return pl.pallas_call(
        _out_body,
        grid=(G,),
        in_specs=[
            pl.BlockSpec((R, D), lambda i: (i, 0)),
            pl.BlockSpec((R, D), lambda i: (i, 0)),
            pl.BlockSpec((R, D), lambda i: (i, 0)),
            pl.BlockSpec((D, D), lambda i: (0, 0)),
            pl.BlockSpec((D, D), lambda i: (0, 0)),
            pl.BlockSpec((1, D), lambda i: (0, 0)),
        ],
        out_specs=pl.BlockSpec((R, D), lambda i: (i, 0)),
        out_shape=jax.ShapeDtypeStruct((N, D), jnp.float32),
    )(a0, a1, x, W_neigh, W_self, b_row)


@jax.jit
def kernel(g, node_feat, edge_feat, embed_weight, W_self, W_neigh, w_gate, b):
    N, D = embed_weight.shape
    E = edge_feat.shape[0]
    src = g[0]
    dst = g[1]
    gate = _compute_gate(edge_feat, w_gate)
    aggs = _make_sc_agg(N, D, E)(src, dst, gate, embed_weight)
    return _compute_out(aggs[0], aggs[1], embed_weight, W_neigh, W_self, b)


# speedup vs baseline: 5.2054x; 1.0193x over previous
import functools

import jax
import jax.numpy as jnp
from jax import lax
from jax.experimental import pallas as pl
from jax.experimental.pallas import tpu as pltpu
from jax.experimental.pallas import tpu_sc as plsc

_NC = 2
_NS = 16
_LANES = 16


def _lane_bcast(v, l):
    idx = jnp.full((_LANES, 1), l, jnp.int32)
    dn = lax.GatherDimensionNumbers(
        offset_dims=(), collapsed_slice_dims=(0,), start_index_map=(0,))
    return lax.gather(v, idx, dn, slice_sizes=(1,),
                      mode=lax.GatherScatterMode.PROMISE_IN_BOUNDS)


def _gate_body(rows_per_step, ef_ref, w_ref, out_ref):
    i = pl.program_id(0)
    z = jnp.sum(ef_ref[...] * w_ref[...], axis=1)
    z2 = jax.nn.sigmoid(z).reshape(rows_per_step, 128)
    out_ref[pl.ds(i * rows_per_step, rows_per_step), :] = z2


def _compute_gate(edge_feat, w_gate):
    E, DE = edge_feat.shape
    P = 128
    R = E // P
    G = 25
    BE = E // G
    w_row = w_gate.reshape(1, DE)
    out = pl.pallas_call(
        functools.partial(_gate_body, BE // P),
        grid=(G,),
        in_specs=[
            pl.BlockSpec((BE, DE), lambda i: (i, 0)),
            pl.BlockSpec((1, DE), lambda i: (0, 0)),
        ],
        out_specs=pl.BlockSpec((R, P), lambda i: (0, 0)),
        out_shape=jax.ShapeDtypeStruct((R, P), jnp.float32),
    )(edge_feat, w_row)
    return out.reshape(E)


def _make_sc_agg(N, D, E):
    NW = _NC * _NS
    EW = E // NW
    C = 80
    CH = EW // C
    RT = 624
    ZR = 48
    TAIL = N - _NS * RT
    GRP = C // _LANES
    JD = D // _LANES
    assert EW * NW == E and CH * C == EW and CH % 2 == 1
    assert RT == 13 * ZR and TAIL == 16

    mesh = plsc.VectorSubcoreMesh(core_axis_name="c", subcore_axis_name="s")

    @functools.partial(
        pl.kernel,
        out_type=jax.ShapeDtypeStruct((_NC, N, D), jnp.float32),
        mesh=mesh,
        scratch_types=[
            pltpu.VMEM((EW,), jnp.int32),
            pltpu.VMEM((EW,), jnp.float32),
            pltpu.VMEM((C,), jnp.int32),
            pltpu.VMEM((C,), jnp.int32),
            pltpu.VMEM((C, D), jnp.float32),
            pltpu.VMEM((C, D), jnp.float32),
            pltpu.VMEM((ZR, D), jnp.float32),
            pltpu.VMEM_SHARED((N, D), jnp.float32),
            pltpu.SemaphoreType.DMA,
            pltpu.SemaphoreType.DMA,
            pltpu.SemaphoreType.DMA,
            pltpu.SemaphoreType.DMA,
            pltpu.SemaphoreType.DMA,
            pltpu.SemaphoreType.DMA,
            pltpu.SemaphoreType.DMA,
        ],
    )
    def sc_agg(src_hbm, dst_hbm, gate_hbm, x_hbm, out_hbm,
               src_all, gate_all, dstv0, dstv1, rows0, rows1,
               zbuf, agg_sh, sem_i, sem_g0, sem_g1, sem_s0, sem_s1,
               sem_d0, sem_d1):
        c = lax.axis_index("c")
        s = lax.axis_index("s")
        wid = s * _NC + c
        base_w = wid * EW

        cp_src = pltpu.async_copy(src_hbm.at[pl.ds(base_w, EW)], src_all, sem_i)
        cp_gate = pltpu.async_copy(gate_hbm.at[pl.ds(base_w, EW)], gate_all, sem_i)

        zeros = jnp.zeros((_LANES,), jnp.float32)

        def zrow(r, carry):
            for j in range(JD):
                zbuf[r, pl.ds(j * _LANES, _LANES)] = zeros
            return carry

        lax.fori_loop(0, ZR, zrow, 0)
        for i in range(RT // ZR):
            pltpu.sync_copy(zbuf, agg_sh.at[pl.ds(s * RT + i * ZR, ZR)])

        @pl.when(s == 0)
        def _zero_tail():
            pltpu.sync_copy(zbuf.at[pl.ds(0, TAIL)], agg_sh.at[pl.ds(_NS * RT, TAIL)])

        cp_src.wait()
        cp_gate.wait()
        plsc.subcore_barrier()

        def start_gather(k, rows, sem):
            pltpu.async_copy(x_hbm.at[src_all.at[pl.ds(k * C, C)]], rows, sem)

        def wait_gather(k, rows, sem):
            pltpu.make_async_copy(x_hbm.at[src_all.at[pl.ds(k * C, C)]], rows, sem).wait()

        def start_dst(k, dstv, sem):
            pltpu.async_copy(dst_hbm.at[pl.ds(base_w + k * C, C)], dstv, sem)

        def wait_dst(k, dstv, sem):
            pltpu.make_async_copy(dst_hbm.at[pl.ds(base_w + k * C, C)], dstv, sem).wait()

        def start_scatter(rows, dstv, sem):
            pltpu.async_copy(rows, agg_sh.at[dstv], sem, add=True)

        def wait_scatter(rows, dstv, sem):
            pltpu.make_async_copy(rows, agg_sh.at[dstv], sem).wait()

        def scale(k, rows):
            def grp(q, gcarry):
                gv = gate_all[pl.ds(k * C + q * _LANES, _LANES)]
                for l in range(_LANES):
                    g16 = _lane_bcast(gv, l)
                    e = q * _LANES + l
                    for j in range(JD):
                        sl = pl.ds(j * _LANES, _LANES)
                        rows[e, sl] = rows[e, sl] * g16
                return gcarry

            lax.fori_loop(0, GRP, grp, 0)

        start_gather(0, rows0, sem_g0)
        start_dst(0, dstv0, sem_d0)
        start_gather(1, rows1, sem_g1)
        start_dst(1, dstv1, sem_d1)
        wait_gather(0, rows0, sem_g0)
        scale(0, rows0)
        wait_dst(0, dstv0, sem_d0)
        start_scatter(rows0, dstv0, sem_s0)

        def pair(t, carry):
            a = 2 * t + 1
            wait_gather(a, rows1, sem_g1)
            scale(a, rows1)
            wait_dst(a, dstv1, sem_d1)
            start_scatter(rows1, dstv1, sem_s1)
            wait_scatter(rows0, dstv0, sem_s0)
            start_gather(a + 1, rows0, sem_g0)
            start_dst(a + 1, dstv0, sem_d0)
            wait_gather(a + 1, rows0, sem_g0)
            scale(a + 1, rows0)
            wait_dst(a + 1, dstv0, sem_d0)
            start_scatter(rows0, dstv0, sem_s0)
            wait_scatter(rows1, dstv1, sem_s1)

            @pl.when(a + 2 < CH)
            def _next():
                start_gather(a + 2, rows1, sem_g1)
                start_dst(a + 2, dstv1, sem_d1)

            return carry

        lax.fori_loop(0, (CH - 1) // 2, pair, 0)
        wait_scatter(rows0, dstv0, sem_s0)
        plsc.subcore_barrier()

        pltpu.sync_copy(agg_sh.at[pl.ds(s * RT, RT)], out_hbm.at[c, pl.ds(s * RT, RT)])

        @pl.when(s == 0)
        def _write_tail():
            pltpu.sync_copy(agg_sh.at[pl.ds(_NS * RT, TAIL)],
                            out_hbm.at[c, pl.ds(_NS * RT, TAIL)])

    return sc_agg


def _out_body(a0_ref, a1_ref, x_ref, wn_ref, ws_ref, b_ref, o_ref):
    agg = a0_ref[...] + a1_ref[...]
    acc = jnp.dot(agg, wn_ref[...], preferred_element_type=jnp.float32)
    acc = acc + jnp.dot(x_ref[...], ws_ref[...], preferred_element_type=jnp.float32)
    acc = acc + b_ref[...]
    o_ref[...] = jnp.maximum(acc, 0.0)


def _compute_out(a0, a1, x, W_neigh, W_self, b):
    N, D = x.shape
    R = 1000
    G = N // R
    b_row = b.reshape(1, D)
    return pl.pallas_call(
        _out_body,
        grid=(G,),
        in_specs=[
            pl.BlockSpec((R, D), lambda i: (i, 0)),
            pl.BlockSpec((R, D), lambda i: (i, 0)),
            pl.BlockSpec((R, D), lambda i: (i, 0)),
            pl.BlockSpec((D, D), lambda i: (0, 0)),
            pl.BlockSpec((D, D), lambda i: (0, 0)),
            pl.BlockSpec((1, D), lambda i: (0, 0)),
        ],
        out_specs=pl.BlockSpec((R, D), lambda i: (i, 0)),
        out_shape=jax.ShapeDtypeStruct((N, D), jnp.float32),
    )(a0, a1, x, W_neigh, W_self, b_row)


@jax.jit
def kernel(g, node_feat, edge_feat, embed_weight, W_self, W_neigh, w_gate, b):
    N, D = embed_weight.shape
    E = edge_feat.shape[0]
    src = g[0]
    dst = g[1]
    gate = _compute_gate(edge_feat, w_gate)
    aggs = _make_sc_agg(N, D, E)(src, dst, gate, embed_weight)
    return _compute_out(aggs[0], aggs[1], embed_weight, W_neigh, W_self, b)


# speedup vs baseline: 6.5175x; 1.2521x over previous
import functools

import jax
import jax.numpy as jnp
from jax import lax
from jax.experimental import pallas as pl
from jax.experimental.pallas import tpu as pltpu
from jax.experimental.pallas import tpu_sc as plsc

_NC = 2
_NS = 16
_LANES = 16


def _lane_bcast(v, l):
    idx = jnp.full((_LANES, 1), l, jnp.int32)
    dn = lax.GatherDimensionNumbers(
        offset_dims=(), collapsed_slice_dims=(0,), start_index_map=(0,))
    return lax.gather(v, idx, dn, slice_sizes=(1,),
                      mode=lax.GatherScatterMode.PROMISE_IN_BOUNDS)


def _gate_body(rows_per_step, ef_ref, w_ref, out_ref):
    i = pl.program_id(0)
    z = jnp.sum(ef_ref[...] * w_ref[...], axis=1)
    z2 = jax.nn.sigmoid(z).reshape(rows_per_step, 128)
    out_ref[pl.ds(i * rows_per_step, rows_per_step), :] = z2


def _compute_gate(edge_feat, w_gate):
    E, DE = edge_feat.shape
    P = 128
    R = E // P
    G = 25
    BE = E // G
    w_row = w_gate.reshape(1, DE)
    out = pl.pallas_call(
        functools.partial(_gate_body, BE // P),
        grid=(G,),
        in_specs=[
            pl.BlockSpec((BE, DE), lambda i: (i, 0)),
            pl.BlockSpec((1, DE), lambda i: (0, 0)),
        ],
        out_specs=pl.BlockSpec((R, P), lambda i: (0, 0)),
        out_shape=jax.ShapeDtypeStruct((R, P), jnp.float32),
    )(edge_feat, w_row)
    return out.reshape(E)


def _make_sc_agg(N, D, E):
    NW = _NC * _NS
    EW = E // NW
    C = 80
    CH = EW // C
    RT = 624
    ZR = 48
    TAIL = N - _NS * RT
    GRP = C // _LANES
    JD = D // _LANES
    assert EW * NW == E and CH * C == EW and CH % 3 == 2 and CH >= 8
    assert RT == 13 * ZR and TAIL == 16

    mesh = plsc.VectorSubcoreMesh(core_axis_name="c", subcore_axis_name="s")

    @functools.partial(
        pl.kernel,
        out_type=jax.ShapeDtypeStruct((_NC, N, D), jnp.float32),
        mesh=mesh,
        scratch_types=[
            pltpu.VMEM((EW,), jnp.float32),
            [pltpu.VMEM((C,), jnp.int32) for _ in range(3)],
            [pltpu.VMEM((C,), jnp.int32) for _ in range(3)],
            [pltpu.VMEM((C, D), jnp.float32) for _ in range(3)],
            pltpu.VMEM((ZR, D), jnp.float32),
            pltpu.VMEM_SHARED((N, D), jnp.float32),
            pltpu.SemaphoreType.DMA,
            [pltpu.SemaphoreType.DMA for _ in range(3)],
            [pltpu.SemaphoreType.DMA for _ in range(3)],
            [pltpu.SemaphoreType.DMA for _ in range(3)],
            [pltpu.SemaphoreType.DMA for _ in range(3)],
        ],
    )
    def sc_agg(src_hbm, dst_hbm, gate_hbm, x_hbm, out_hbm,
               gate_all, srcv, dstv, rows, zbuf, agg_sh,
               sem_i, sem_src, sem_dst, sem_g, sem_s):
        c = lax.axis_index("c")
        s = lax.axis_index("s")
        wid = s * _NC + c
        base_w = wid * EW

        cp_gate = pltpu.async_copy(gate_hbm.at[pl.ds(base_w, EW)], gate_all, sem_i)

        zeros = jnp.zeros((_LANES,), jnp.float32)

        def zrow(r, carry):
            for j in range(JD):
                zbuf[r, pl.ds(j * _LANES, _LANES)] = zeros
            return carry

        lax.fori_loop(0, ZR, zrow, 0)
        for i in range(RT // ZR):
            pltpu.sync_copy(zbuf, agg_sh.at[pl.ds(s * RT + i * ZR, ZR)])

        @pl.when(s == 0)
        def _zero_tail():
            pltpu.sync_copy(zbuf.at[pl.ds(0, TAIL)], agg_sh.at[pl.ds(_NS * RT, TAIL)])

        cp_gate.wait()
        plsc.subcore_barrier()

        def start_src(k, p):
            pltpu.async_copy(src_hbm.at[pl.ds(base_w + k * C, C)], srcv[p], sem_src[p])

        def wait_src(k, p):
            pltpu.make_async_copy(src_hbm.at[pl.ds(base_w + k * C, C)],
                                  srcv[p], sem_src[p]).wait()

        def start_dst(k, p):
            pltpu.async_copy(dst_hbm.at[pl.ds(base_w + k * C, C)], dstv[p], sem_dst[p])

        def wait_dst(k, p):
            pltpu.make_async_copy(dst_hbm.at[pl.ds(base_w + k * C, C)],
                                  dstv[p], sem_dst[p]).wait()

        def start_gather(p):
            pltpu.async_copy(x_hbm.at[srcv[p]], rows[p], sem_g[p])

        def wait_gather(p):
            pltpu.make_async_copy(x_hbm.at[srcv[p]], rows[p], sem_g[p]).wait()

        def start_scatter(p):
            pltpu.async_copy(rows[p], agg_sh.at[dstv[p]], sem_s[p], add=True)

        def wait_scatter(p):
            pltpu.make_async_copy(rows[p], agg_sh.at[dstv[p]], sem_s[p]).wait()

        def scale(k, p):
            def grp(q, gcarry):
                gv = gate_all[pl.ds(k * C + q * _LANES, _LANES)]
                for l in range(_LANES):
                    g16 = _lane_bcast(gv, l)
                    e = q * _LANES + l
                    for j in range(JD):
                        sl = pl.ds(j * _LANES, _LANES)
                        rows[p][e, sl] = rows[p][e, sl] * g16
                return gcarry

            lax.fori_loop(0, GRP, grp, 0)

        for j in range(3):
            start_src(j, j)
            start_dst(j, j)
        for j in range(3):
            wait_src(j, j)
            start_gather(j)

        wait_gather(0)
        scale(0, 0)
        wait_dst(0, 0)
        start_scatter(0)
        start_src(3, 0)
        wait_gather(1)
        scale(1, 1)
        wait_dst(1, 1)
        start_scatter(1)

        def full_step(k, p, pn, pn2, with_next2):
            wait_scatter(pn)
            start_dst(k + 1, pn)
            if with_next2:
                start_src(k + 2, pn2)
            wait_src(k + 1, pn)
            start_gather(pn)
            wait_gather(p)
            scale(k, p)
            wait_dst(k, p)
            start_scatter(p)

        def triple(t, carry):
            k0 = 3 * t + 2
            full_step(k0, 2, 0, 1, True)
            full_step(k0 + 1, 0, 1, 2, True)
            full_step(k0 + 2, 1, 2, 0, True)
            return carry

        lax.fori_loop(0, (CH - 5) // 3, triple, 0)
        full_step(CH - 3, 2, 0, 1, True)
        full_step(CH - 2, 0, 1, 2, False)
        wait_scatter(2)
        wait_gather(1)
        scale(CH - 1, 1)
        wait_dst(CH - 1, 1)
        start_scatter(1)
        wait_scatter(0)
        wait_scatter(1)
        plsc.subcore_barrier()

        pltpu.sync_copy(agg_sh.at[pl.ds(s * RT, RT)], out_hbm.at[c, pl.ds(s * RT, RT)])

        @pl.when(s == 0)
        def _write_tail():
            pltpu.sync_copy(agg_sh.at[pl.ds(_NS * RT, TAIL)],
                            out_hbm.at[c, pl.ds(_NS * RT, TAIL)])

    return sc_agg


def _out_body(a0_ref, a1_ref, x_ref, wn_ref, ws_ref, b_ref, o_ref):
    agg = a0_ref[...] + a1_ref[...]
    acc = jnp.dot(agg, wn_ref[...], preferred_element_type=jnp.float32)
    acc = acc + jnp.dot(x_ref[...], ws_ref[...], preferred_element_type=jnp.float32)
    acc = acc + b_ref[...]
    o_ref[...] = jnp.maximum(acc, 0.0)


def _compute_out(a0, a1, x, W_neigh, W_self, b):
    N, D = x.shape
    R = 1000
    G = N // R
    b_row = b.reshape(1, D)
    return pl.pallas_call(
        _out_body,
        grid=(G,),
        in_specs=[
            pl.BlockSpec((R, D), lambda i: (i, 0)),
            pl.BlockSpec((R, D), lambda i: (i, 0)),
            pl.BlockSpec((R, D), lambda i: (i, 0)),
            pl.BlockSpec((D, D), lambda i: (0, 0)),
            pl.BlockSpec((D, D), lambda i: (0, 0)),
            pl.BlockSpec((1, D), lambda i: (0, 0)),
        ],
        out_specs=pl.BlockSpec((R, D), lambda i: (i, 0)),
        out_shape=jax.ShapeDtypeStruct((N, D), jnp.float32),
    )(a0, a1, x, W_neigh, W_self, b_row)


@jax.jit
def kernel(g, node_feat, edge_feat, embed_weight, W_self, W_neigh, w_gate, b):
    N, D = embed_weight.shape
    E = edge_feat.shape[0]
    src = g[0]
    dst = g[1]
    gate = _compute_gate(edge_feat, w_gate)
    aggs = _make_sc_agg(N, D, E)(src, dst, gate, embed_weight)
    return _compute_out(aggs[0], aggs[1], embed_weight, W_neigh, W_self, b)


# speedup vs baseline: 6.5916x; 1.0114x over previous
import functools

import jax
import jax.numpy as jnp
from jax import lax
from jax.experimental import pallas as pl
from jax.experimental.pallas import tpu as pltpu
from jax.experimental.pallas import tpu_sc as plsc

_NC = 2
_NS = 16
_LANES = 16


def _lane_bcast(v, l):
    idx = jnp.full((_LANES, 1), l, jnp.int32)
    dn = lax.GatherDimensionNumbers(
        offset_dims=(), collapsed_slice_dims=(0,), start_index_map=(0,))
    return lax.gather(v, idx, dn, slice_sizes=(1,),
                      mode=lax.GatherScatterMode.PROMISE_IN_BOUNDS)


def _gate_body(ef_ref, w_ref, out_ref):
    z = jnp.dot(ef_ref[...], w_ref[...], preferred_element_type=jnp.float32)
    out_ref[...] = jax.nn.sigmoid(z)


def _compute_gate(edge_feat, w_gate):
    E, DE = edge_feat.shape
    P = 128
    U = E // 8
    G = 25
    BR = U // G
    ef128 = edge_feat.reshape(U, P)
    wrep = jnp.tile(jnp.kron(jnp.eye(8, dtype=jnp.float32),
                             w_gate.reshape(DE, 1)), (1, 16))
    out = pl.pallas_call(
        _gate_body,
        grid=(G,),
        in_specs=[
            pl.BlockSpec((BR, P), lambda i: (i, 0)),
            pl.BlockSpec((P, P), lambda i: (0, 0)),
        ],
        out_specs=pl.BlockSpec((BR, P), lambda i: (i, 0)),
        out_shape=jax.ShapeDtypeStruct((U, P), jnp.float32),
    )(ef128, wrep)
    return out.reshape(U * P)


def _make_sc_agg(N, D, E):
    NW = _NC * _NS
    EW = E // NW
    C = 80
    CH = EW // C
    RT = 624
    ZR = 48
    TAIL = N - _NS * RT
    GRP = C // _LANES
    JD = D // _LANES
    assert EW * NW == E and CH * C == EW and CH % 3 == 2 and CH >= 8
    assert RT == 13 * ZR and TAIL == 16

    mesh = plsc.VectorSubcoreMesh(core_axis_name="c", subcore_axis_name="s")

    @functools.partial(
        pl.kernel,
        out_type=jax.ShapeDtypeStruct((_NC, N, D), jnp.float32),
        mesh=mesh,
        scratch_types=[
            [pltpu.VMEM((16 * C,), jnp.float32) for _ in range(3)],
            [pltpu.VMEM((C,), jnp.int32) for _ in range(3)],
            [pltpu.VMEM((C,), jnp.int32) for _ in range(3)],
            [pltpu.VMEM((C, D), jnp.float32) for _ in range(3)],
            pltpu.VMEM((ZR, D), jnp.float32),
            pltpu.VMEM_SHARED((N, D), jnp.float32),
            [pltpu.SemaphoreType.DMA for _ in range(3)],
            [pltpu.SemaphoreType.DMA for _ in range(3)],
            [pltpu.SemaphoreType.DMA for _ in range(3)],
            [pltpu.SemaphoreType.DMA for _ in range(3)],
            [pltpu.SemaphoreType.DMA for _ in range(3)],
        ],
    )
    def sc_agg(src_hbm, dst_hbm, gate_hbm, x_hbm, out_hbm,
               gatev, srcv, dstv, rows, zbuf, agg_sh,
               sem_gt, sem_src, sem_dst, sem_g, sem_s):
        c = lax.axis_index("c")
        s = lax.axis_index("s")
        wid = s * _NC + c
        base_w = wid * EW

        zeros = jnp.zeros((_LANES,), jnp.float32)

        def zrow(r, carry):
            for j in range(JD):
                zbuf[r, pl.ds(j * _LANES, _LANES)] = zeros
            return carry

        lax.fori_loop(0, ZR, zrow, 0)
        for i in range(RT // ZR):
            pltpu.sync_copy(zbuf, agg_sh.at[pl.ds(s * RT + i * ZR, ZR)])

        @pl.when(s == 0)
        def _zero_tail():
            pltpu.sync_copy(zbuf.at[pl.ds(0, TAIL)], agg_sh.at[pl.ds(_NS * RT, TAIL)])

        plsc.subcore_barrier()

        def start_src(k, p):
            pltpu.async_copy(src_hbm.at[pl.ds(base_w + k * C, C)], srcv[p], sem_src[p])

        def wait_src(k, p):
            pltpu.make_async_copy(src_hbm.at[pl.ds(base_w + k * C, C)],
                                  srcv[p], sem_src[p]).wait()

        def start_dst(k, p):
            pltpu.async_copy(dst_hbm.at[pl.ds(base_w + k * C, C)], dstv[p], sem_dst[p])

        def wait_dst(k, p):
            pltpu.make_async_copy(dst_hbm.at[pl.ds(base_w + k * C, C)],
                                  dstv[p], sem_dst[p]).wait()

        def start_gate(k, p):
            pltpu.async_copy(gate_hbm.at[pl.ds(16 * (base_w + k * C), 16 * C)],
                             gatev[p], sem_gt[p])

        def wait_gate(k, p):
            pltpu.make_async_copy(gate_hbm.at[pl.ds(16 * (base_w + k * C), 16 * C)],
                                  gatev[p], sem_gt[p]).wait()

        def start_gather(p):
            pltpu.async_copy(x_hbm.at[srcv[p]], rows[p], sem_g[p])

        def wait_gather(p):
            pltpu.make_async_copy(x_hbm.at[srcv[p]], rows[p], sem_g[p]).wait()

        def start_scatter(p):
            pltpu.async_copy(rows[p], agg_sh.at[dstv[p]], sem_s[p], add=True)

        def wait_scatter(p):
            pltpu.make_async_copy(rows[p], agg_sh.at[dstv[p]], sem_s[p]).wait()

        def scale(p):
            def grp(q, gcarry):
                ga = gatev[p][pl.ds(q * 256, _LANES)]
                gb = gatev[p][pl.ds(q * 256 + 128, _LANES)]
                for l in range(_LANES):
                    g16 = _lane_bcast(ga if l < 8 else gb, l % 8)
                    e = q * _LANES + l
                    for j in range(JD):
                        sl = pl.ds(j * _LANES, _LANES)
                        rows[p][e, sl] = rows[p][e, sl] * g16
                return gcarry

            lax.fori_loop(0, GRP, grp, 0)

        for j in range(3):
            start_src(j, j)
            start_dst(j, j)
            start_gate(j, j)
        for j in range(3):
            wait_src(j, j)
            start_gather(j)

        wait_gather(0)
        wait_gate(0, 0)
        scale(0)
        wait_dst(0, 0)
        start_scatter(0)
        start_src(3, 0)
        wait_gather(1)
        wait_gate(1, 1)
        scale(1)
        wait_dst(1, 1)
        start_scatter(1)

        def full_step(k, p, pn, pn2, with_next2):
            wait_scatter(pn)
            start_dst(k + 1, pn)
            start_gate(k + 1, pn)
            if with_next2:
                start_src(k + 2, pn2)
            wait_src(k + 1, pn)
            start_gather(pn)
            wait_gather(p)
            wait_gate(k, p)
            scale(p)
            wait_dst(k, p)
            start_scatter(p)

        def triple(t, carry):
            k0 = 3 * t + 2
            full_step(k0, 2, 0, 1, True)
            full_step(k0 + 1, 0, 1, 2, True)
            full_step(k0 + 2, 1, 2, 0, True)
            return carry

        lax.fori_loop(0, (CH - 5) // 3, triple, 0)
        full_step(CH - 3, 2, 0, 1, True)
        full_step(CH - 2, 0, 1, 2, False)
        wait_scatter(2)
        wait_gather(1)
        wait_gate(CH - 1, 1)
        scale(1)
        wait_dst(CH - 1, 1)
        start_scatter(1)
        wait_scatter(0)
        wait_scatter(1)
        plsc.subcore_barrier()

        pltpu.sync_copy(agg_sh.at[pl.ds(s * RT, RT)], out_hbm.at[c, pl.ds(s * RT, RT)])

        @pl.when(s == 0)
        def _write_tail():
            pltpu.sync_copy(agg_sh.at[pl.ds(_NS * RT, TAIL)],
                            out_hbm.at[c, pl.ds(_NS * RT, TAIL)])

    return sc_agg


def _out_body(a0_ref, a1_ref, x_ref, wn_ref, ws_ref, b_ref, o_ref):
    agg = a0_ref[...] + a1_ref[...]
    acc = jnp.dot(agg, wn_ref[...], preferred_element_type=jnp.float32)
    acc = acc + jnp.dot(x_ref[...], ws_ref[...], preferred_element_type=jnp.float32)
    acc = acc + b_ref[...]
    o_ref[...] = jnp.maximum(acc, 0.0)


def _compute_out(a0, a1, x, W_neigh, W_self, b):
    N, D = x.shape
    R = 1000
    G = N // R
    b_row = b.reshape(1, D)
    return pl.pallas_call(
        _out_body,
        grid=(G,),
        in_specs=[
            pl.BlockSpec((R, D), lambda i: (i, 0)),
            pl.BlockSpec((R, D), lambda i: (i, 0)),
            pl.BlockSpec((R, D), lambda i: (i, 0)),
            pl.BlockSpec((D, D), lambda i: (0, 0)),
            pl.BlockSpec((D, D), lambda i: (0, 0)),
            pl.BlockSpec((1, D), lambda i: (0, 0)),
        ],
        out_specs=pl.BlockSpec((R, D), lambda i: (i, 0)),
        out_shape=jax.ShapeDtypeStruct((N, D), jnp.float32),
    )(a0, a1, x, W_neigh, W_self, b_row)


@jax.jit
def kernel(g, node_feat, edge_feat, embed_weight, W_self, W_neigh, w_gate, b):
    N, D = embed_weight.shape
    E = edge_feat.shape[0]
    src = g[0]
    dst = g[1]
    gate = _compute_gate(edge_feat, w_gate)
    aggs = _make_sc_agg(N, D, E)(src, dst, gate, embed_weight)
    return _compute_out(aggs[0], aggs[1], embed_weight, W_neigh, W_self, b)
